# baseline (speedup 1.0000x reference)
import jax
import jax.numpy as jnp
from functools import partial
from jax.experimental import pallas as pl

B, T, OUT_DIM = 512, 20, 8
MEM, KEY, HID, KNN = 20, 256, 256, 4
F32 = jnp.float32


def _im2col(x, kh, kw, stride, pad):
    n, h, w, c = x.shape
    oh = (h + 2 * pad - kh) // stride + 1
    ow = (w + 2 * pad - kw) // stride + 1
    xp = jnp.pad(x, ((0, 0), (pad, pad), (pad, pad), (0, 0)))
    cols = []
    for ky in range(kh):
        for kx in range(kw):
            cols.append(xp[:, ky:ky + stride * oh:stride, kx:kx + stride * ow:stride, :])
    pat = jnp.stack(cols, axis=3)
    return pat.reshape(n * oh * ow, kh * kw * c), oh, ow


def _mm_relu_kernel(p_ref, w_ref, b_ref, o_ref):
    o_ref[...] = jnp.maximum(
        jnp.dot(p_ref[...], w_ref[...], preferred_element_type=F32) + b_ref[...], 0.0)


def _mm_relu(p, w, b, mt):
    m, k = p.shape
    n = w.shape[1]
    return pl.pallas_call(
        _mm_relu_kernel,
        grid=(m // mt,),
        in_specs=[
            pl.BlockSpec((mt, k), lambda i: (i, 0)),
            pl.BlockSpec((k, n), lambda i: (0, 0)),
            pl.BlockSpec((1, n), lambda i: (0, 0)),
        ],
        out_specs=pl.BlockSpec((mt, n), lambda i: (i, 0)),
        out_shape=jax.ShapeDtypeStruct((m, n), F32),
    )(p, w, b)


def _fc_norm_kernel(h_ref, w1_ref, b1_ref, w2_ref, b2_ref, g_ref, be_ref, o_ref):
    f1 = jnp.maximum(jnp.dot(h_ref[...], w1_ref[...], preferred_element_type=F32) + b1_ref[...], 0.0)
    f2 = jnp.maximum(jnp.dot(f1, w2_ref[...], preferred_element_type=F32) + b2_ref[...], 0.0)
    nb = o_ref.shape[0]
    f = f2.reshape(nb, T, 128)
    mu = jnp.mean(f, axis=1, keepdims=True)
    d = f - mu
    var = jnp.sum(d * d, axis=1, keepdims=True) / (T - 1)
    sigma = jnp.sqrt(var + 1e-08)
    o_ref[...] = d / sigma * g_ref[...] + be_ref[...]


def _fc_norm(h3, w1, b1, w2, b2, gamma, beta, ex_t):
    return pl.pallas_call(
        _fc_norm_kernel,
        grid=(B // ex_t,),
        in_specs=[
            pl.BlockSpec((ex_t * T, 512), lambda i: (i, 0)),
            pl.BlockSpec((512, 256), lambda i: (0, 0)),
            pl.BlockSpec((1, 256), lambda i: (0, 0)),
            pl.BlockSpec((256, 128), lambda i: (0, 0)),
            pl.BlockSpec((1, 128), lambda i: (0, 0)),
            pl.BlockSpec((1, 128), lambda i: (0, 0)),
            pl.BlockSpec((1, 128), lambda i: (0, 0)),
        ],
        out_specs=pl.BlockSpec((ex_t, T, 128), lambda i: (i, 0, 0)),
        out_shape=jax.ShapeDtypeStruct((B, T, 128), F32),
    )(h3, w1, b1.reshape(1, 256), w2, b2.reshape(1, 128),
      gamma.reshape(1, 128), beta.reshape(1, 128))


def _rec_kernel(feat_ref, wih_ref, whh_ref, lb_ref, kw_ref, ww_ref, ew_ref,
                ow_ref, ob_ref, m0_ref, y_ref):
    bt = feat_ref.shape[0]
    wih = wih_ref[...]
    whh = whh_ref[...]
    lb = lb_ref[...]
    kw = kw_ref[...]
    wwt = ww_ref[...]
    ewt = ew_ref[...]
    m_init = jnp.broadcast_to(m0_ref[...][None], (bt, MEM, KEY))
    z = jnp.zeros((bt, HID), F32)

    def step(t, carry):
        h, c, r, M = carry
        ft = feat_ref[:, pl.ds(t, 1), :].reshape(bt, 128)
        inp = jnp.concatenate([ft, r], axis=1)
        gates = (jnp.dot(inp, wih, preferred_element_type=F32)
                 + jnp.dot(h, whh, preferred_element_type=F32) + lb)
        ig = jax.nn.sigmoid(gates[:, :HID])
        fg = jax.nn.sigmoid(gates[:, HID:2 * HID])
        gg = jnp.tanh(gates[:, 2 * HID:3 * HID])
        og = jax.nn.sigmoid(gates[:, 3 * HID:])
        c = fg * c + ig * gg
        h = og * jnp.tanh(c)
        k = jnp.dot(h, kw, preferred_element_type=F32)
        kn = k / (jnp.sqrt(jnp.sum(k * k, axis=1, keepdims=True)) + 1e-08)
        mnorm = jnp.sqrt(jnp.sum(M * M, axis=2, keepdims=True)) + 1e-08
        sims = jnp.sum(kn[:, None, :] * (M / mnorm), axis=2)
        s = sims
        v1 = jnp.max(s, axis=1, keepdims=True)
        s = jnp.where(s >= v1, -1e30, s)
        v2 = jnp.max(s, axis=1, keepdims=True)
        s = jnp.where(s >= v2, -1e30, s)
        v3 = jnp.max(s, axis=1, keepdims=True)
        s = jnp.where(s >= v3, -1e30, s)
        v4 = jnp.max(s, axis=1, keepdims=True)
        sel = (sims >= v4).astype(F32)
        e = jnp.exp(sims - v1) * sel
        w = e / jnp.sum(e, axis=1, keepdims=True)
        r = jnp.sum(w[:, :, None] * M, axis=1)
        wv = jnp.tanh(jnp.dot(h, wwt, preferred_element_type=F32))
        ev = jax.nn.sigmoid(jnp.dot(h, ewt, preferred_element_type=F32))
        e2 = jnp.exp(sims - v1)
        wwr = e2 / jnp.sum(e2, axis=1, keepdims=True)
        M = M * (1.0 - wwr[:, :, None] * ev[:, None, :]) + wwr[:, :, None] * wv[:, None, :]
        return h, c, r, M

    h, c, r, M = jax.lax.fori_loop(0, T, step, (z, z, z, m_init))
    y_ref[...] = (jnp.dot(jnp.concatenate([h, r], axis=1), ow_ref[...],
                          preferred_element_type=F32) + ob_ref[...])


def _recurrence(feat, wih_t, whh_t, lstm_b, kw_t, ww_t, ew_t, ow_t, out_b, mem_init, bt):
    return pl.pallas_call(
        _rec_kernel,
        grid=(B // bt,),
        in_specs=[
            pl.BlockSpec((bt, T, 128), lambda i: (i, 0, 0)),
            pl.BlockSpec((384, 4 * HID), lambda i: (0, 0)),
            pl.BlockSpec((HID, 4 * HID), lambda i: (0, 0)),
            pl.BlockSpec((1, 4 * HID), lambda i: (0, 0)),
            pl.BlockSpec((HID, KEY), lambda i: (0, 0)),
            pl.BlockSpec((HID, KEY), lambda i: (0, 0)),
            pl.BlockSpec((HID, KEY), lambda i: (0, 0)),
            pl.BlockSpec((HID + KEY, OUT_DIM), lambda i: (0, 0)),
            pl.BlockSpec((1, OUT_DIM), lambda i: (0, 0)),
            pl.BlockSpec((MEM, KEY), lambda i: (0, 0)),
        ],
        out_specs=pl.BlockSpec((bt, OUT_DIM), lambda i: (i, 0)),
        out_shape=jax.ShapeDtypeStruct((B, OUT_DIM), F32),
    )(feat, wih_t, whh_t, lstm_b.reshape(1, 4 * HID), kw_t, ww_t, ew_t,
      ow_t, out_b.reshape(1, OUT_DIM), mem_init)


@partial(jax.jit, static_argnames=('device',))
def _run(x, conv1_w, conv1_b, conv2_w, conv2_b, conv3_w, conv3_b, fc1_w,
         fc1_b, fc2_w, fc2_b, gamma, beta, lstm_wih, lstm_whh, lstm_b,
         key_w, write_w, erase_w, out_w, out_b, mem_init, device=0):
    n = B * T
    xi = x.reshape(n, 32, 32, 1)

    p1, oh, ow = _im2col(xi, 4, 4, 2, 1)
    w1 = conv1_w.transpose(2, 3, 1, 0).reshape(16, 32)
    h1 = _mm_relu(p1, w1, conv1_b.reshape(1, 32), 4096).reshape(n, 16, 16, 32)

    p2, oh, ow = _im2col(h1, 4, 4, 2, 1)
    w2 = conv2_w.transpose(2, 3, 1, 0).reshape(512, 32)
    h2 = _mm_relu(p2, w2, conv2_b.reshape(1, 32), 1024).reshape(n, 8, 8, 32)

    p3, oh, ow = _im2col(h2, 4, 4, 2, 1)
    w3 = conv3_w.transpose(2, 3, 1, 0).reshape(512, 32)
    h3 = _mm_relu(p3, w3, conv3_b.reshape(1, 32), 1024).reshape(n, 512)

    w_fc1 = fc1_w.reshape(256, 32, 4, 4).transpose(2, 3, 1, 0).reshape(512, 256)
    feat = _fc_norm(h3, w_fc1, fc1_b, fc2_w.T, fc2_b, gamma, beta, 8)

    y = _recurrence(feat, lstm_wih.T, lstm_whh.T, lstm_b, key_w.T,
                    write_w.T, erase_w.T, out_w.T, out_b, mem_init, 128)
    return y, jnp.argmax(y, axis=1)


def kernel(x, conv1_w, conv1_b, conv2_w, conv2_b, conv3_w, conv3_b, fc1_w,
           fc1_b, fc2_w, fc2_b, gamma, beta, lstm_wih, lstm_whh, lstm_b,
           key_w, write_w, erase_w, out_w, out_b, mem_init, device=0):
    return _run(x, conv1_w, conv1_b, conv2_w, conv2_b, conv3_w, conv3_b,
                fc1_w, fc1_b, fc2_w, fc2_b, gamma, beta, lstm_wih,
                lstm_whh, lstm_b, key_w, write_w, erase_w, out_w, out_b,
                mem_init)


# speedup vs baseline: 622.4760x; 622.4760x over previous
import numpy as np
import jax
import jax.numpy as jnp
from functools import partial
from jax.experimental import pallas as pl
from jax.experimental.pallas import tpu as pltpu

B, T, OUT_DIM = 512, 20, 8
MEM, KEY, HID, KNN = 20, 256, 256, 4
F32 = jnp.float32


def _sel(nx, nox):
    s = np.zeros((nx, 4, nox), np.float32)
    for kx in range(4):
        for ox in range(nox):
            x = 2 * ox + kx - 1
            if 0 <= x < nx:
                s[x, kx, ox] = 1.0
    return s


_SEL1 = _sel(32, 16)
_SEL2 = _sel(16, 8)
_SEL3 = _sel(8, 4)


def _conv_stage(h, nt, oy, wky_ref, bias_ref):
    l = h.shape[2]
    lout = wky_ref.shape[2]
    hp = jnp.pad(h, ((0, 0), (1, 1), (0, 0)))
    h4 = hp.reshape(nt, oy + 1, 2, l)
    acc = None
    for ky in range(4):
        rows = h4[:, ky // 2:ky // 2 + oy, ky & 1, :]
        g = jnp.dot(rows.reshape(nt * oy, l), wky_ref[ky],
                    preferred_element_type=F32)
        acc = g if acc is None else acc + g
    return jnp.maximum(acc + bias_ref[...], 0.0).reshape(nt, oy, lout)


def _enc_kernel(x_ref, w1_ref, b1_ref, w2_ref, b2_ref, w3_ref, b3_ref,
                f1_ref, f1b_ref, f2_ref, f2b_ref, o_ref):
    nt = x_ref.shape[0]
    h1 = _conv_stage(x_ref[...], nt, 16, w1_ref, b1_ref)
    h2 = _conv_stage(h1, nt, 8, w2_ref, b2_ref)
    h3 = _conv_stage(h2, nt, 4, w3_ref, b3_ref)
    flat = jnp.concatenate([h3[:, y, :] for y in range(4)], axis=1)
    f1 = jnp.maximum(jnp.dot(flat, f1_ref[...], preferred_element_type=F32)
                     + f1b_ref[...], 0.0)
    o_ref[...] = jnp.maximum(jnp.dot(f1, f2_ref[...],
                                     preferred_element_type=F32)
                             + f2b_ref[...], 0.0)


def _encoder(x2d, w1s, b1, w2s, b2, w3s, b3, f1, f1b, f2, f2b, nt):
    n = B * T
    return pl.pallas_call(
        _enc_kernel,
        grid=(n // nt,),
        in_specs=[
            pl.BlockSpec((nt, 32, 32), lambda i: (i, 0, 0)),
            pl.BlockSpec((4, 32, 512), lambda i: (0, 0, 0)),
            pl.BlockSpec((1, 512), lambda i: (0, 0)),
            pl.BlockSpec((4, 512, 256), lambda i: (0, 0, 0)),
            pl.BlockSpec((1, 256), lambda i: (0, 0)),
            pl.BlockSpec((4, 256, 128), lambda i: (0, 0, 0)),
            pl.BlockSpec((1, 128), lambda i: (0, 0)),
            pl.BlockSpec((512, 256), lambda i: (0, 0)),
            pl.BlockSpec((1, 256), lambda i: (0, 0)),
            pl.BlockSpec((256, 128), lambda i: (0, 0)),
            pl.BlockSpec((1, 128), lambda i: (0, 0)),
        ],
        out_specs=pl.BlockSpec((nt, 128), lambda i: (i, 0)),
        out_shape=jax.ShapeDtypeStruct((n, 128), F32),
    )(x2d, w1s, b1, w2s, b2, w3s, b3, f1, f1b, f2, f2b)


def _rec_kernel(feat_ref, g_ref, be_ref, wih_ref, whh_ref, lb_ref, kw_ref,
                ww_ref, ew_ref, ow_ref, ob_ref, m0_ref, y_ref, fn_ref):
    bt = feat_ref.shape[0]
    f = feat_ref[...]
    mu = jnp.mean(f, axis=1, keepdims=True)
    d = f - mu
    var = jnp.sum(d * d, axis=1, keepdims=True) / (T - 1)
    fn_ref[...] = d / jnp.sqrt(var + 1e-08) * g_ref[...] + be_ref[...]

    wih = wih_ref[...]
    whh = whh_ref[...]
    lb = lb_ref[...]
    kw = kw_ref[...]
    wwt = ww_ref[...]
    ewt = ew_ref[...]
    m_init = jnp.broadcast_to(m0_ref[...][None], (bt, MEM, KEY))
    z = jnp.zeros((bt, HID), F32)

    def step(t, carry):
        h, c, r, M = carry
        ft = fn_ref[:, pl.ds(t, 1), :].reshape(bt, 128)
        inp = jnp.concatenate([ft, r], axis=1)
        gates = (jnp.dot(inp, wih, preferred_element_type=F32)
                 + jnp.dot(h, whh, preferred_element_type=F32) + lb)
        ig = jax.nn.sigmoid(gates[:, :HID])
        fg = jax.nn.sigmoid(gates[:, HID:2 * HID])
        gg = jnp.tanh(gates[:, 2 * HID:3 * HID])
        og = jax.nn.sigmoid(gates[:, 3 * HID:])
        c = fg * c + ig * gg
        h = og * jnp.tanh(c)
        k = jnp.dot(h, kw, preferred_element_type=F32)
        kn = k / (jnp.sqrt(jnp.sum(k * k, axis=1, keepdims=True)) + 1e-08)
        mnorm = jnp.sqrt(jnp.sum(M * M, axis=2, keepdims=True)) + 1e-08
        sims = jnp.sum(kn[:, None, :] * (M / mnorm), axis=2)
        s = sims
        v1 = jnp.max(s, axis=1, keepdims=True)
        s = jnp.where(s >= v1, -1e30, s)
        v2 = jnp.max(s, axis=1, keepdims=True)
        s = jnp.where(s >= v2, -1e30, s)
        v3 = jnp.max(s, axis=1, keepdims=True)
        s = jnp.where(s >= v3, -1e30, s)
        v4 = jnp.max(s, axis=1, keepdims=True)
        sel = (sims >= v4).astype(F32)
        e = jnp.exp(sims - v1) * sel
        w = e / jnp.sum(e, axis=1, keepdims=True)
        r = jnp.sum(w[:, :, None] * M, axis=1)
        wv = jnp.tanh(jnp.dot(h, wwt, preferred_element_type=F32))
        ev = jax.nn.sigmoid(jnp.dot(h, ewt, preferred_element_type=F32))
        e2 = jnp.exp(sims - v1)
        wwr = e2 / jnp.sum(e2, axis=1, keepdims=True)
        M = M * (1.0 - wwr[:, :, None] * ev[:, None, :]) \
            + wwr[:, :, None] * wv[:, None, :]
        return h, c, r, M

    h, c, r, M = jax.lax.fori_loop(0, T, step, (z, z, z, m_init))
    y_ref[...] = (jnp.dot(jnp.concatenate([h, r], axis=1), ow_ref[...],
                          preferred_element_type=F32) + ob_ref[...])


def _recurrence(feat, gamma, beta, wih_t, whh_t, lstm_b, kw_t, ww_t, ew_t,
                ow_t, out_b, mem_init, bt):
    return pl.pallas_call(
        _rec_kernel,
        grid=(B // bt,),
        in_specs=[
            pl.BlockSpec((bt, T, 128), lambda i: (i, 0, 0)),
            pl.BlockSpec((1, 1, 128), lambda i: (0, 0, 0)),
            pl.BlockSpec((1, 1, 128), lambda i: (0, 0, 0)),
            pl.BlockSpec((384, 4 * HID), lambda i: (0, 0)),
            pl.BlockSpec((HID, 4 * HID), lambda i: (0, 0)),
            pl.BlockSpec((1, 4 * HID), lambda i: (0, 0)),
            pl.BlockSpec((HID, KEY), lambda i: (0, 0)),
            pl.BlockSpec((HID, KEY), lambda i: (0, 0)),
            pl.BlockSpec((HID, KEY), lambda i: (0, 0)),
            pl.BlockSpec((HID + KEY, OUT_DIM), lambda i: (0, 0)),
            pl.BlockSpec((1, OUT_DIM), lambda i: (0, 0)),
            pl.BlockSpec((MEM, KEY), lambda i: (0, 0)),
        ],
        out_specs=pl.BlockSpec((bt, OUT_DIM), lambda i: (i, 0)),
        out_shape=jax.ShapeDtypeStruct((B, OUT_DIM), F32),
        scratch_shapes=[pltpu.VMEM((bt, T, 128), F32)],
    )(feat, gamma.reshape(1, 1, 128), beta.reshape(1, 1, 128), wih_t, whh_t,
      lstm_b.reshape(1, 4 * HID), kw_t, ww_t, ew_t, ow_t,
      out_b.reshape(1, OUT_DIM), mem_init)


@partial(jax.jit, static_argnames=('device',))
def _run(x, conv1_w, conv1_b, conv2_w, conv2_b, conv3_w, conv3_b, fc1_w,
         fc1_b, fc2_w, fc2_b, gamma, beta, lstm_wih, lstm_whh, lstm_b,
         key_w, write_w, erase_w, out_w, out_b, mem_init, device=0):
    x2d = x.reshape(B * T, 32, 32)

    w1c = conv1_w.transpose(2, 3, 1, 0)[:, :, 0, :]
    w1s = jnp.stack([jnp.einsum('xko,kc->xoc', _SEL1, w1c[ky])
                     .reshape(32, 512) for ky in range(4)])
    w2c = conv2_w.transpose(2, 3, 1, 0)
    w2s = jnp.stack([jnp.einsum('xko,kcd->xcod', _SEL2, w2c[ky])
                     .reshape(512, 256) for ky in range(4)])
    w3c = conv3_w.transpose(2, 3, 1, 0)
    w3s = jnp.stack([jnp.einsum('xko,kcd->xcod', _SEL3, w3c[ky])
                     .reshape(256, 128) for ky in range(4)])
    b1 = jnp.tile(conv1_b, 16).reshape(1, 512)
    b2 = jnp.tile(conv2_b, 8).reshape(1, 256)
    b3 = jnp.tile(conv3_b, 4).reshape(1, 128)
    f1p = fc1_w.reshape(256, 32, 4, 4).transpose(2, 3, 1, 0).reshape(512, 256)

    feat = _encoder(x2d, w1s, b1, w2s, b2, w3s, b3, f1p,
                    fc1_b.reshape(1, 256), fc2_w.T, fc2_b.reshape(1, 128), 64)

    y = _recurrence(feat.reshape(B, T, 128), gamma, beta, lstm_wih.T,
                    lstm_whh.T, lstm_b, key_w.T, write_w.T, erase_w.T,
                    out_w.T, out_b, mem_init, 128)
    return y, jnp.argmax(y, axis=1)


def kernel(x, conv1_w, conv1_b, conv2_w, conv2_b, conv3_w, conv3_b, fc1_w,
           fc1_b, fc2_w, fc2_b, gamma, beta, lstm_wih, lstm_whh, lstm_b,
           key_w, write_w, erase_w, out_w, out_b, mem_init, device=0):
    return _run(x, conv1_w, conv1_b, conv2_w, conv2_b, conv3_w, conv3_b,
                fc1_w, fc1_b, fc2_w, fc2_b, gamma, beta, lstm_wih,
                lstm_whh, lstm_b, key_w, write_w, erase_w, out_w, out_b,
                mem_init)


# speedup vs baseline: 623.7160x; 1.0020x over previous
import numpy as np
import jax
import jax.numpy as jnp
from functools import partial
from jax.experimental import pallas as pl
from jax.experimental.pallas import tpu as pltpu

B, T, OUT_DIM = 512, 20, 8
MEM, KEY, HID, KNN = 20, 256, 256, 4
F32 = jnp.float32


def _sel(nx, nox):
    s = np.zeros((nx, 4, nox), np.float32)
    for kx in range(4):
        for ox in range(nox):
            x = 2 * ox + kx - 1
            if 0 <= x < nx:
                s[x, kx, ox] = 1.0
    return s


_SEL1 = _sel(32, 16)
_SEL2 = _sel(16, 8)
_SEL3 = _sel(8, 4)


def _pack(h, nt, y, l):
    hp = jnp.pad(h.reshape(nt, y, l), ((0, 0), (1, 1), (0, 0)))
    u = y // 2 + 1
    hp4 = hp.reshape(nt, u, 2, l)
    return jnp.concatenate([hp4[:, :, 0, :], hp4[:, :, 1, :]], axis=2)


def _conv_stage(hpk, nt, oy, wp_ref, bias_ref):
    rows4 = jnp.concatenate([hpk[:, 0:oy, :], hpk[:, 1:oy + 1, :]], axis=2)
    g = jnp.dot(rows4.reshape(nt * oy, rows4.shape[2]), wp_ref[...],
                preferred_element_type=F32)
    return jnp.maximum(g + bias_ref[...], 0.0)


def _enc_kernel(x_ref, w1_ref, b1_ref, w2_ref, b2_ref, w3_ref, b3_ref,
                f1_ref, f1b_ref, f2_ref, f2b_ref, o_ref):
    nt = x_ref.shape[0]
    h1 = _conv_stage(x_ref[...], nt, 16, w1_ref, b1_ref)
    h2 = _conv_stage(_pack(h1, nt, 16, 512), nt, 8, w2_ref, b2_ref)
    h3 = _conv_stage(_pack(h2, nt, 8, 256), nt, 4, w3_ref, b3_ref)
    h3v = h3.reshape(nt, 4, 128)
    flat = jnp.concatenate([h3v[:, y, :] for y in range(4)], axis=1)
    f1 = jnp.maximum(jnp.dot(flat, f1_ref[...], preferred_element_type=F32)
                     + f1b_ref[...], 0.0)
    o_ref[...] = jnp.maximum(jnp.dot(f1, f2_ref[...],
                                     preferred_element_type=F32)
                             + f2b_ref[...], 0.0)


def _encoder(xpk, w1s, b1, w2s, b2, w3s, b3, f1, f1b, f2, f2b, nt):
    n = B * T
    return pl.pallas_call(
        _enc_kernel,
        grid=(n // nt,),
        in_specs=[
            pl.BlockSpec((nt, 17, 64), lambda i: (i, 0, 0)),
            pl.BlockSpec((128, 512), lambda i: (0, 0)),
            pl.BlockSpec((1, 512), lambda i: (0, 0)),
            pl.BlockSpec((2048, 256), lambda i: (0, 0)),
            pl.BlockSpec((1, 256), lambda i: (0, 0)),
            pl.BlockSpec((1024, 128), lambda i: (0, 0)),
            pl.BlockSpec((1, 128), lambda i: (0, 0)),
            pl.BlockSpec((512, 256), lambda i: (0, 0)),
            pl.BlockSpec((1, 256), lambda i: (0, 0)),
            pl.BlockSpec((256, 128), lambda i: (0, 0)),
            pl.BlockSpec((1, 128), lambda i: (0, 0)),
        ],
        out_specs=pl.BlockSpec((nt, 128), lambda i: (i, 0)),
        out_shape=jax.ShapeDtypeStruct((n, 128), F32),
    )(xpk, w1s, b1, w2s, b2, w3s, b3, f1, f1b, f2, f2b)


def _rec_kernel(feat_ref, g_ref, be_ref, wih_ref, whh_ref, lb_ref, kw_ref,
                ww_ref, ew_ref, ow_ref, ob_ref, m0_ref, y_ref, fn_ref):
    bt = feat_ref.shape[0]
    f = feat_ref[...]
    mu = jnp.mean(f, axis=1, keepdims=True)
    d = f - mu
    var = jnp.sum(d * d, axis=1, keepdims=True) / (T - 1)
    fn_ref[...] = d / jnp.sqrt(var + 1e-08) * g_ref[...] + be_ref[...]

    wih = wih_ref[...]
    whh = whh_ref[...]
    lb = lb_ref[...]
    kw = kw_ref[...]
    wwt = ww_ref[...]
    ewt = ew_ref[...]
    m_init = jnp.broadcast_to(m0_ref[...][None], (bt, MEM, KEY))
    z = jnp.zeros((bt, HID), F32)

    def step(t, carry):
        h, c, r, M = carry
        ft = fn_ref[:, pl.ds(t, 1), :].reshape(bt, 128)
        inp = jnp.concatenate([ft, r], axis=1)
        gates = (jnp.dot(inp, wih, preferred_element_type=F32)
                 + jnp.dot(h, whh, preferred_element_type=F32) + lb)
        ig = jax.nn.sigmoid(gates[:, :HID])
        fg = jax.nn.sigmoid(gates[:, HID:2 * HID])
        gg = jnp.tanh(gates[:, 2 * HID:3 * HID])
        og = jax.nn.sigmoid(gates[:, 3 * HID:])
        c = fg * c + ig * gg
        h = og * jnp.tanh(c)
        k = jnp.dot(h, kw, preferred_element_type=F32)
        kn = k / (jnp.sqrt(jnp.sum(k * k, axis=1, keepdims=True)) + 1e-08)
        mnorm = jnp.sqrt(jnp.sum(M * M, axis=2, keepdims=True)) + 1e-08
        sims = jnp.sum(kn[:, None, :] * (M / mnorm), axis=2)
        s = sims
        v1 = jnp.max(s, axis=1, keepdims=True)
        s = jnp.where(s >= v1, -1e30, s)
        v2 = jnp.max(s, axis=1, keepdims=True)
        s = jnp.where(s >= v2, -1e30, s)
        v3 = jnp.max(s, axis=1, keepdims=True)
        s = jnp.where(s >= v3, -1e30, s)
        v4 = jnp.max(s, axis=1, keepdims=True)
        sel = (sims >= v4).astype(F32)
        e = jnp.exp(sims - v1) * sel
        w = e / jnp.sum(e, axis=1, keepdims=True)
        r = jnp.sum(w[:, :, None] * M, axis=1)
        wv = jnp.tanh(jnp.dot(h, wwt, preferred_element_type=F32))
        ev = jax.nn.sigmoid(jnp.dot(h, ewt, preferred_element_type=F32))
        e2 = jnp.exp(sims - v1)
        wwr = e2 / jnp.sum(e2, axis=1, keepdims=True)
        M = M * (1.0 - wwr[:, :, None] * ev[:, None, :]) \
            + wwr[:, :, None] * wv[:, None, :]
        return h, c, r, M

    h, c, r, M = jax.lax.fori_loop(0, T, step, (z, z, z, m_init))
    y_ref[...] = (jnp.dot(jnp.concatenate([h, r], axis=1), ow_ref[...],
                          preferred_element_type=F32) + ob_ref[...])


def _recurrence(feat, gamma, beta, wih_t, whh_t, lstm_b, kw_t, ww_t, ew_t,
                ow_t, out_b, mem_init, bt):
    return pl.pallas_call(
        _rec_kernel,
        grid=(B // bt,),
        in_specs=[
            pl.BlockSpec((bt, T, 128), lambda i: (i, 0, 0)),
            pl.BlockSpec((1, 1, 128), lambda i: (0, 0, 0)),
            pl.BlockSpec((1, 1, 128), lambda i: (0, 0, 0)),
            pl.BlockSpec((384, 4 * HID), lambda i: (0, 0)),
            pl.BlockSpec((HID, 4 * HID), lambda i: (0, 0)),
            pl.BlockSpec((1, 4 * HID), lambda i: (0, 0)),
            pl.BlockSpec((HID, KEY), lambda i: (0, 0)),
            pl.BlockSpec((HID, KEY), lambda i: (0, 0)),
            pl.BlockSpec((HID, KEY), lambda i: (0, 0)),
            pl.BlockSpec((HID + KEY, OUT_DIM), lambda i: (0, 0)),
            pl.BlockSpec((1, OUT_DIM), lambda i: (0, 0)),
            pl.BlockSpec((MEM, KEY), lambda i: (0, 0)),
        ],
        out_specs=pl.BlockSpec((bt, OUT_DIM), lambda i: (i, 0)),
        out_shape=jax.ShapeDtypeStruct((B, OUT_DIM), F32),
        scratch_shapes=[pltpu.VMEM((bt, T, 128), F32)],
    )(feat, gamma.reshape(1, 1, 128), beta.reshape(1, 1, 128), wih_t, whh_t,
      lstm_b.reshape(1, 4 * HID), kw_t, ww_t, ew_t, ow_t,
      out_b.reshape(1, OUT_DIM), mem_init)


@partial(jax.jit, static_argnames=('device',))
def _run(x, conv1_w, conv1_b, conv2_w, conv2_b, conv3_w, conv3_b, fc1_w,
         fc1_b, fc2_w, fc2_b, gamma, beta, lstm_wih, lstm_whh, lstm_b,
         key_w, write_w, erase_w, out_w, out_b, mem_init, device=0):
    xpk = jnp.pad(x.reshape(B * T, 32, 32),
                  ((0, 0), (1, 1), (0, 0))).reshape(B * T, 17, 64)

    w1c = conv1_w.transpose(2, 3, 1, 0)[:, :, 0, :]
    w1s = jnp.concatenate([jnp.einsum('xko,kc->xoc', _SEL1, w1c[ky])
                           .reshape(32, 512) for ky in range(4)], axis=0)
    w2c = conv2_w.transpose(2, 3, 1, 0)
    w2s = jnp.concatenate([jnp.einsum('xko,kcd->xcod', _SEL2, w2c[ky])
                           .reshape(512, 256) for ky in range(4)], axis=0)
    w3c = conv3_w.transpose(2, 3, 1, 0)
    w3s = jnp.concatenate([jnp.einsum('xko,kcd->xcod', _SEL3, w3c[ky])
                           .reshape(256, 128) for ky in range(4)], axis=0)
    b1 = jnp.tile(conv1_b, 16).reshape(1, 512)
    b2 = jnp.tile(conv2_b, 8).reshape(1, 256)
    b3 = jnp.tile(conv3_b, 4).reshape(1, 128)
    f1p = fc1_w.reshape(256, 32, 4, 4).transpose(2, 3, 1, 0).reshape(512, 256)

    feat = _encoder(xpk, w1s, b1, w2s, b2, w3s, b3, f1p,
                    fc1_b.reshape(1, 256), fc2_w.T, fc2_b.reshape(1, 128), 64)

    y = _recurrence(feat.reshape(B, T, 128), gamma, beta, lstm_wih.T,
                    lstm_whh.T, lstm_b, key_w.T, write_w.T, erase_w.T,
                    out_w.T, out_b, mem_init, 128)
    return y, jnp.argmax(y, axis=1)


def kernel(x, conv1_w, conv1_b, conv2_w, conv2_b, conv3_w, conv3_b, fc1_w,
           fc1_b, fc2_w, fc2_b, gamma, beta, lstm_wih, lstm_whh, lstm_b,
           key_w, write_w, erase_w, out_w, out_b, mem_init, device=0):
    return _run(x, conv1_w, conv1_b, conv2_w, conv2_b, conv3_w, conv3_b,
                fc1_w, fc1_b, fc2_w, fc2_b, gamma, beta, lstm_wih,
                lstm_whh, lstm_b, key_w, write_w, erase_w, out_w, out_b,
                mem_init)
